# baseline (speedup 1.0000x reference)
import dataclasses
import functools

import jax
import jax.numpy as jnp
import numpy as np
from jax import lax
from jax.experimental import pallas as pl
from jax.experimental.pallas import tpu as pltpu
from jax.experimental.pallas import tpu_sc as plsc

N = 10000
E = 320000
D = 128

NC = 2
NS = 16
NW = NC * NS
EPT = E // NW

DC = 125
DNCH = EPT // DC

C = 64
E2 = 327680
EPT2 = E2 // NW
NCHT = EPT2 // C
GRP = 40
NG = NCHT // GRP
NBG = 4
NBS = 2
RB = 64
NP = 10240
RPT = NP // NS

_PA = np.array([32 * (w // 16) + w % 16 for w in range(D // 2)], np.int64)
_PB = _PA + 16

_mesh = plsc.VectorSubcoreMesh(core_axis_name="c", subcore_axis_name="s")

_sc_params = pltpu.CompilerParams()
for _f, _v in (("needs_layout_passes", False), ("use_tc_tiling_on_sc", False)):
    if _f in pltpu.CompilerParams.__dataclass_fields__:
        _sc_params = dataclasses.replace(_sc_params, **{_f: _v})


def _sc_degree(dst3d, ew3d):
    @functools.partial(
        pl.kernel,
        out_type=jax.ShapeDtypeStruct((NC * NP,), jnp.float32),
        mesh=_mesh,
        scratch_types=[
            pltpu.VMEM((DNCH, DC), jnp.int32),
            pltpu.VMEM((DNCH, DC), jnp.float32),
            pltpu.VMEM((2048,), jnp.float32),
            pltpu.VMEM_SHARED((NP,), jnp.float32),
            pltpu.SemaphoreType.DMA,
        ],
    )
    def k(dst_hbm, ew_hbm, out_hbm, didx, wbuf, zbuf, acc, sem):
        cid = lax.axis_index("c")
        sid = lax.axis_index("s")
        wid = sid * NC + cid

        pltpu.sync_copy(dst_hbm.at[wid], didx)
        pltpu.sync_copy(ew_hbm.at[wid], wbuf)

        @pl.loop(0, 2048 // 16)
        def _(i):
            zbuf[pl.ds(i * 16, 16)] = jnp.zeros((16,), jnp.float32)

        @pl.when(sid == 0)
        def _():
            @pl.loop(0, NP // 2048)
            def _(j):
                pltpu.sync_copy(zbuf, acc.at[pl.ds(j * 2048, 2048)])

        plsc.subcore_barrier()

        @pl.loop(0, DNCH // 16)
        def _(gq):
            for kk in range(16):
                pltpu.async_copy(wbuf.at[gq * 16 + kk],
                                 acc.at[didx.at[gq * 16 + kk]], sem,
                                 add=True)
            for kk in range(16):
                pltpu.make_async_copy(wbuf.at[gq * 16 + kk],
                                      acc.at[didx.at[gq * 16 + kk]],
                                      sem).wait()

        plsc.subcore_barrier()

        @pl.when(sid == 0)
        def _():
            pltpu.sync_copy(acc, out_hbm.at[pl.ds(cid * NP, NP)])

    return k(dst3d, ew3d)


def _sc_scatter(yb, src3d, dst3d, ewp3d):
    @functools.partial(
        pl.kernel,
        out_type=jax.ShapeDtypeStruct((NC, NP, D), jnp.float32),
        mesh=_mesh,
        scratch_types=[
            pltpu.VMEM((GRP, C), jnp.int32),
            pltpu.VMEM((GRP, C), jnp.int32),
            pltpu.VMEM((GRP, C), jnp.float32),
            pltpu.VMEM((RB, D // 2), jnp.uint32),
            pltpu.VMEM((RB, D // 2), jnp.uint32),
            pltpu.VMEM((RB, D // 2), jnp.uint32),
            pltpu.VMEM((RB, D // 2), jnp.uint32),
            pltpu.VMEM((RB, D), jnp.float32),
            pltpu.VMEM((RB, D), jnp.float32),
            pltpu.VMEM_SHARED((NP, D), jnp.float32),
            pltpu.SemaphoreType.DMA,
            pltpu.SemaphoreType.DMA,
            pltpu.SemaphoreType.DMA,
            pltpu.SemaphoreType.DMA,
            pltpu.SemaphoreType.DMA,
            pltpu.SemaphoreType.DMA,
            pltpu.SemaphoreType.DMA,
        ],
        compiler_params=_sc_params,
    )
    def k(yb_hbm, src_hbm, dst_hbm, ewp_hbm, out_hbm,
          sidx, didx, wbuf, gb0, gb1, gb2, gb3, sb0, sb1, acc,
          g0, g1, g2, g3, s0, s1, zsem):
        cid = lax.axis_index("c")
        sid = lax.axis_index("s")
        wid = sid * NC + cid
        gbuf = (gb0, gb1, gb2, gb3)
        sbuf = (sb0, sb1)
        gsem = (g0, g1, g2, g3)
        ssem = (s0, s1)

        def gather(i, b):
            pltpu.async_copy(yb_hbm.at[sidx.at[i]], gbuf[b], gsem[b])

        def gather_wait(i, b):
            pltpu.make_async_copy(yb_hbm.at[sidx.at[i]], gbuf[b],
                                  gsem[b]).wait()

        def scatter(i, b):
            pltpu.async_copy(sbuf[b], acc.at[didx.at[i]], ssem[b], add=True)

        def scatter_wait(i, b):
            pltpu.make_async_copy(sbuf[b], acc.at[didx.at[i]],
                                  ssem[b]).wait()

        @pl.loop(0, RB)
        def _(r):
            for j in range(D // 16):
                sb0[r, pl.ds(j * 16, 16)] = jnp.zeros((16,), jnp.float32)

        for t in range(RPT // RB):
            pltpu.async_copy(sb0, acc.at[pl.ds(sid * RPT + t * RB, RB)], zsem)
        for t in range(RPT // RB):
            pltpu.make_async_copy(
                sb0, acc.at[pl.ds(sid * RPT + t * RB, RB)], zsem).wait()

        plsc.subcore_barrier()

        @pl.loop(0, NG)
        def _(gg):
            pltpu.sync_copy(src_hbm.at[wid, pl.ds(gg * GRP, GRP)], sidx)
            pltpu.sync_copy(dst_hbm.at[wid, pl.ds(gg * GRP, GRP)], didx)
            pltpu.sync_copy(ewp_hbm.at[wid, pl.ds(gg * GRP, GRP)], wbuf)

            for b in range(NBG):
                gather(b, b)

            @pl.loop(0, GRP // NBG)
            def _(q):
                for j in range(NBG):
                    gb = j
                    sb = j % NBS
                    i = q * NBG + j
                    gather_wait(i, gb)

                    if j < NBS:
                        @pl.when(q > 0)
                        def _():
                            scatter_wait(q * NBG + j - NBS, sb)
                    else:
                        scatter_wait(i - NBS, sb)

                    @pl.loop(0, RB // 16)
                    def _(cv):
                        off = pl.multiple_of(cv * 16, 16)
                        w16 = wbuf[i, pl.ds(off, 16)]
                        for l in range(16):
                            wv = jnp.full((16,), w16[l], jnp.float32)
                            r = cv * 16 + l
                            for jj in range(D // 32):
                                words = gbuf[gb][r, pl.ds(jj * 16, 16)]
                                pr = plsc.bitcast(words, jnp.bfloat16)
                                ua, ub = plsc.unpack(
                                    pr, format=plsc.PackFormat.INTERLEAVED)
                                sbuf[sb][r, pl.ds(jj * 32, 16)] = ua * wv
                                sbuf[sb][r, pl.ds(jj * 32 + 16, 16)] = ub * wv

                    scatter(i, sb)

                    @pl.when(q < GRP // NBG - 1)
                    def _():
                        gather(i + NBG, gb)

            scatter_wait(GRP - 2, (GRP - 2) % NBS)
            scatter_wait(GRP - 1, (GRP - 1) % NBS)

        plsc.subcore_barrier()

        pltpu.sync_copy(acc.at[pl.ds(sid * RPT, RPT)],
                        out_hbm.at[cid, pl.ds(sid * RPT, RPT)])

    return k(yb, src3d, dst3d, ewp3d)


_ROWS_BLK = 1000
_GRID = N // _ROWS_BLK


def _rows_spec():
    return pl.BlockSpec((_ROWS_BLK, D), lambda i: (i, 0))


def _full_spec(shape):
    return pl.BlockSpec(shape, lambda i: tuple(0 for _ in shape))


def _pack_bf16_pairs(a, b):
    a32 = lax.bitcast_convert_type(a, jnp.uint32)
    b32 = lax.bitcast_convert_type(b, jnp.uint32)
    ab = (a32 + 0x7FFF + ((a32 >> 16) & 1)) >> 16
    bb = (b32 + 0x7FFF + ((b32 >> 16) & 1)) >> 16
    return ab | (bb << 16)


def _half_spec():
    return pl.BlockSpec((_ROWS_BLK, D // 2), lambda i: (i, 0))


def _tc_first(x, W, WA, WB, degT):
    def body(x_ref, w_ref, wa_ref, wb_ref, dp_ref, ys_ref, yb_ref, dis_ref):
        deg = dp_ref[...][:, 0:1] + dp_ref[...][:, 1:2] + 1.0
        dis = jnp.broadcast_to(lax.rsqrt(deg), ys_ref.shape)
        h = x_ref[...]
        ys_ref[...] = dis * jnp.dot(h, w_ref[...],
                                    preferred_element_type=jnp.float32)
        dh = dis[:, :D // 2]
        A = dh * jnp.dot(h, wa_ref[...], preferred_element_type=jnp.float32)
        B = dh * jnp.dot(h, wb_ref[...], preferred_element_type=jnp.float32)
        yb_ref[...] = _pack_bf16_pairs(A, B)
        dis_ref[...] = dis

    return pl.pallas_call(
        body,
        grid=(_GRID,),
        in_specs=[_rows_spec(), _full_spec((D, D)), _full_spec((D, D // 2)),
                  _full_spec((D, D // 2)),
                  pl.BlockSpec((_ROWS_BLK, 2), lambda i: (i, 0))],
        out_specs=[_rows_spec(), _half_spec(), _rows_spec()],
        out_shape=[jax.ShapeDtypeStruct((N, D), jnp.float32),
                   jax.ShapeDtypeStruct((N, D // 2), jnp.uint32),
                   jax.ShapeDtypeStruct((N, D), jnp.float32)],
    )(x, W, WA, WB, degT)


def _tc_mid(p0, p1, ys, dis, b, W, WA, WB):
    def body(p0_ref, p1_ref, ys_ref, dis_ref, b_ref, w_ref, wa_ref, wb_ref,
             oys_ref, oyb_ref):
        t = dis_ref[...] * (p0_ref[...] + p1_ref[...] + ys_ref[...]) + b_ref[...]
        h = jnp.where(t > 0, t, jnp.exp(t) - 1.0)
        dis = dis_ref[...]
        oys_ref[...] = dis * jnp.dot(h, w_ref[...],
                                     preferred_element_type=jnp.float32)
        dh = dis[:, :D // 2]
        A = dh * jnp.dot(h, wa_ref[...], preferred_element_type=jnp.float32)
        B = dh * jnp.dot(h, wb_ref[...], preferred_element_type=jnp.float32)
        oyb_ref[...] = _pack_bf16_pairs(A, B)

    return pl.pallas_call(
        body,
        grid=(_GRID,),
        in_specs=[_rows_spec(), _rows_spec(), _rows_spec(), _rows_spec(),
                  _full_spec((1, D)), _full_spec((D, D)),
                  _full_spec((D, D // 2)), _full_spec((D, D // 2))],
        out_specs=[_rows_spec(), _half_spec()],
        out_shape=[jax.ShapeDtypeStruct((N, D), jnp.float32),
                   jax.ShapeDtypeStruct((N, D // 2), jnp.uint32)],
    )(p0, p1, ys, dis, b, W, WA, WB)


def _tc_last(p0, p1, ys, dis, b):
    def body(p0_ref, p1_ref, ys_ref, dis_ref, b_ref, o_ref):
        t = dis_ref[...] * (p0_ref[...] + p1_ref[...] + ys_ref[...]) + b_ref[...]
        nrm = jnp.sqrt(jnp.sum(t * t, axis=1, keepdims=True))
        o_ref[...] = t / jnp.maximum(nrm, 1e-12)

    return pl.pallas_call(
        body,
        grid=(_GRID,),
        in_specs=[_rows_spec(), _rows_spec(), _rows_spec(), _rows_spec(),
                  _full_spec((1, D))],
        out_specs=_rows_spec(),
        out_shape=jax.ShapeDtypeStruct((N, D), jnp.float32),
    )(p0, p1, ys, dis, b)


def kernel(x, edge_index, edge_weight, W1, b1, W2, b2, W3, b3):
    pad_idx = (jnp.arange(E2 - E, dtype=jnp.int32) % N).astype(jnp.int32)
    src3d = jnp.concatenate([edge_index[0], pad_idx]).reshape(NW, NCHT, C)
    dst3d = jnp.concatenate([edge_index[1], pad_idx]).reshape(NW, NCHT, C)
    ewp3d = jnp.concatenate(
        [edge_weight, jnp.zeros((E2 - E,), jnp.float32)]).reshape(NW, NCHT, C)

    pa = jnp.asarray(_PA)
    pb = jnp.asarray(_PB)

    degp = _sc_degree(edge_index[1].reshape(NW, DNCH, DC),
                      edge_weight.reshape(NW, DNCH, DC)).reshape(NC, NP)

    ys, yb, dis = _tc_first(x, W1, W1[:, pa], W1[:, pb], degp.T)
    for b, Wn in ((b1, W2), (b2, W3)):
        p = _sc_scatter(yb, src3d, dst3d, ewp3d)
        ys, yb = _tc_mid(p[0], p[1], ys, dis, b.reshape(1, D),
                         Wn, Wn[:, pa], Wn[:, pb])
    p = _sc_scatter(yb, src3d, dst3d, ewp3d)
    return _tc_last(p[0], p[1], ys, dis, b3.reshape(1, D))
